# baseline (speedup 1.0000x reference)
import functools

import jax
import jax.numpy as jnp
from jax import lax
from jax.experimental import pallas as pl
from jax.experimental.pallas import tpu as pltpu
from jax.experimental.pallas import tpu_sc as plsc

_VOCAB = 1000000
_D = 32
_OUT = 1000
_B = 16384
_H = 200

_NC = 2
_NS = 16
_NW = _NC * _NS

_TCOLS = _VOCAB // 128
_TAIL = _VOCAB - _TCOLS * 128


_NDB = _TCOLS // 2
_BW = 256


def _transpose_body(tabt_hbm, out_hbm, inb, outc, isem, osem):
    wid = lax.axis_index("s") * _NC + lax.axis_index("c")
    n_t = jnp.where(wid < _NDB - 32 * (_NDB // 32), _NDB // 32 + 1,
                    _NDB // 32)
    iota16 = lax.iota(jnp.int32, 16)
    cvecs = [iota16 + 16 * k for k in range(8)]
    svecs = [iota16 * 32 + 512 * k for k in range(8)]

    def in_copy(t, p):
        c = wid + 32 * jnp.minimum(t, n_t - 1)
        c0 = pl.multiple_of(c * _BW, 128)
        for r in range(4):
            pltpu.async_copy(
                tabt_hbm.at[pl.ds(8 * r, 8), pl.ds(c0, _BW)],
                inb.at[p, r], isem)

    def wait_in(p):
        for r in range(4):
            pltpu.make_async_copy(
                tabt_hbm.at[pl.ds(0, 8), pl.ds(0, _BW)],
                inb.at[p, r], isem).wait()

    def out_copy(t, p):
        c = wid + 32 * t
        pltpu.async_copy(outc.at[p],
                         out_hbm.at[pl.ds(c * (_BW * _D), _BW * _D)], osem)

    def wait_out(p):
        pltpu.make_async_copy(outc.at[0],
                              out_hbm.at[pl.ds(0, _BW * _D)], osem).wait()

    def transpose_block(p, nks):
        idx_p = jnp.full((16,), p, dtype=jnp.int32)
        for s, nk in enumerate(nks):

            def diag_body(d0, carry, s=s, nk=nk):
                dvec = (d0 + iota16) & 31
                d8 = dvec >> 3
                d7 = dvec & 7
                vs = [plsc.load_gather(
                    inb, [idx_p, d8, d7, cvecs[k] + 128 * s])
                    for k in range(nk)]
                for k in range(nk):
                    plsc.store_scatter(
                        outc, [idx_p, svecs[k] + dvec + 4096 * s], vs[k])
                return carry

            lax.fori_loop(0, 32, diag_body, 0)

    in_copy(0, 0)
    in_copy(1, 1)
    wait_in(0)
    transpose_block(0, (8, 8))
    in_copy(2, 0)
    out_copy(0, 0)
    wait_in(1)
    transpose_block(1, (8, 8))
    in_copy(3, 1)
    out_copy(1, 1)

    def steady(t, carry):
        p = t % 2
        wait_in(p)
        wait_out(p)
        transpose_block(p, (8, 8))
        in_copy(t + 2, p)
        out_copy(t, p)
        return carry

    lax.fori_loop(2, n_t, steady, 0)
    wait_out(0)
    wait_out(1)
    wait_in(0)
    wait_in(1)

    @pl.when(wid == _NW - 1)
    def _tail():
        tail_c0 = pl.multiple_of(
            lax.max(wid * 128, jnp.int32((_TCOLS - 1) * 128)), 128)
        for r in range(4):
            pltpu.sync_copy(
                tabt_hbm.at[pl.ds(8 * r, 8), pl.ds(tail_c0, _BW)],
                inb.at[0, r])
        transpose_block(0, (8, _TAIL // 16))
        n_tail = (128 + _TAIL) * _D
        pltpu.sync_copy(outc.at[0, pl.ds(0, n_tail)],
                        out_hbm.at[pl.ds((_TCOLS - 1) * 128 * _D, n_tail)])


def _transpose_table(tabt):
    mesh = plsc.VectorSubcoreMesh(core_axis_name="c", subcore_axis_name="s")
    return pl.kernel(
        _transpose_body,
        out_type=jax.ShapeDtypeStruct((_VOCAB * _D,), jnp.float32),
        mesh=mesh,
        scratch_types=[
            pltpu.VMEM((2, 4, 8, _BW), jnp.float32),
            pltpu.VMEM((2, _BW * _D), jnp.float32),
            pltpu.SemaphoreType.DMA,
            pltpu.SemaphoreType.DMA,
        ],
        compiler_params=pltpu.CompilerParams(use_tc_tiling_on_sc=True,
                                             needs_layout_passes=False),
    )(tabt)


_BH = _B // 2
_RPW = _BH // _NW
_C = 8
_G1 = 104
_G2 = _H - _G1
_NCHUNK = _RPW // _C
_NPAIR = _NCHUNK // 2


def _fire_gathers(tab, idx_b, rows_b, sem):
    for c in range(_C):
        pltpu.async_copy(
            tab.at[idx_b.at[c, pl.ds(0, _G1)]],
            rows_b.at[pl.ds(c * _H, _G1)], sem)
        pltpu.async_copy(
            tab.at[idx_b.at[c, pl.ds(_G1, _G2)]],
            rows_b.at[pl.ds(c * _H + _G1, _G2)], sem)


def _wait_gathers(tab, idx_b, rows_b, sem):
    for c in range(_C):
        pltpu.make_async_copy(
            tab.at[idx_b.at[c, pl.ds(0, _G1)]],
            rows_b.at[pl.ds(c * _H, _G1)], sem).wait()
        pltpu.make_async_copy(
            tab.at[idx_b.at[c, pl.ds(_G1, _G2)]],
            rows_b.at[pl.ds(c * _H + _G1, _G2)], sem).wait()


def _wait_idx(x_hbm, idx_b, sem):
    pltpu.make_async_copy(x_hbm.at[pl.ds(0, _C)], idx_b, sem).wait()


def _reduce_chunk(rows_b, stage_v, chunk):
    inv = jnp.float32(1.0 / _H)
    for c in range(_C):
        s0 = c * _H

        def body(j, accs, s0=s0):
            a0, a1, b0, b1 = accs
            r = s0 + 4 * j
            a0 = a0 + rows_b[r, pl.ds(0, 16)]
            a1 = a1 + rows_b[r, pl.ds(16, 16)]
            b0 = b0 + rows_b[r + 1, pl.ds(0, 16)]
            b1 = b1 + rows_b[r + 1, pl.ds(16, 16)]
            a0 = a0 + rows_b[r + 2, pl.ds(0, 16)]
            a1 = a1 + rows_b[r + 2, pl.ds(16, 16)]
            b0 = b0 + rows_b[r + 3, pl.ds(0, 16)]
            b1 = b1 + rows_b[r + 3, pl.ds(16, 16)]
            return (a0, a1, b0, b1)

        z = jnp.zeros((16,), jnp.float32)
        a0, a1, b0, b1 = lax.fori_loop(0, _H // 4, body, (z, z, z, z))
        slot = chunk * _C + c
        stage_v[slot, pl.ds(0, 16)] = (a0 + b0) * inv
        stage_v[slot, pl.ds(16, 16)] = (a1 + b1) * inv


def _pool_body(half, x_hbm, tab_hbm, out_hbm,
               idx0, idx1, rows0, rows1, stage_v, isem, gsem0, gsem1):
    wid = lax.axis_index("s") * _NC + lax.axis_index("c")
    row0 = half * _BH + wid * _RPW

    def idx_copy(t, dst):
        base = row0 + jnp.minimum(t, _NCHUNK - 1) * _C
        pltpu.async_copy(x_hbm.at[pl.ds(base, _C)], dst, isem)

    idx_copy(0, idx0)
    _wait_idx(x_hbm, idx0, isem)
    _fire_gathers(tab_hbm, idx0, rows0, gsem0)
    idx_copy(1, idx1)

    def pair_body(p, carry):
        a = 2 * p
        _wait_idx(x_hbm, idx1, isem)
        _fire_gathers(tab_hbm, idx1, rows1, gsem1)
        _wait_gathers(tab_hbm, idx0, rows0, gsem0)
        idx_copy(a + 2, idx0)
        _reduce_chunk(rows0, stage_v, a)
        _wait_idx(x_hbm, idx0, isem)
        _fire_gathers(tab_hbm, idx0, rows0, gsem0)
        _wait_gathers(tab_hbm, idx1, rows1, gsem1)
        idx_copy(a + 3, idx1)
        _reduce_chunk(rows1, stage_v, a + 1)
        return carry

    lax.fori_loop(0, _NPAIR, pair_body, 0)

    _wait_idx(x_hbm, idx1, isem)
    _wait_gathers(tab_hbm, idx0, rows0, gsem0)
    pltpu.sync_copy(stage_v, out_hbm.at[pl.ds(wid * _RPW, _RPW)])


def _pool(x, tab, half):
    mesh = plsc.VectorSubcoreMesh(core_axis_name="c", subcore_axis_name="s")
    return pl.kernel(
        functools.partial(_pool_body, half),
        out_type=jax.ShapeDtypeStruct((_BH, _D), jnp.float32),
        mesh=mesh,
        scratch_types=[
            pltpu.VMEM((_C, _H), jnp.int32),
            pltpu.VMEM((_C, _H), jnp.int32),
            pltpu.VMEM((_C * _H, _D), jnp.float32),
            pltpu.VMEM((_C * _H, _D), jnp.float32),
            pltpu.VMEM((_RPW, _D), jnp.float32),
            pltpu.SemaphoreType.DMA,
            pltpu.SemaphoreType.DMA,
            pltpu.SemaphoreType.DMA,
        ],
        compiler_params=pltpu.CompilerParams(use_tc_tiling_on_sc=False),
    )(x, tab)


_BM = 2048
_NBLK = _BH // _BM


def _mm_body(w_ref, p_ref, b_ref, o_ref):
    o_ref[...] = (
        lax.dot_general(
            w_ref[...], p_ref[...],
            dimension_numbers=(((1,), (1,)), ((), ())),
            preferred_element_type=jnp.float32,
        )
        + b_ref[...]
    )


def _mm_body2(w_ref, p_ref, b_ref, prev_ref, o_ref):
    del prev_ref
    _mm_body(w_ref, p_ref, b_ref, o_ref)


def _matmul_t(pooled1, pooled2, w, bias):
    part = pl.pallas_call(
        _mm_body,
        grid=(_NBLK,),
        in_specs=[
            pl.BlockSpec((_OUT, _D), lambda i: (0, 0)),
            pl.BlockSpec((_BM, _D), lambda i: (i, 0)),
            pl.BlockSpec((_OUT, 1), lambda i: (0, 0)),
        ],
        out_specs=pl.BlockSpec((_OUT, _BM), lambda i: (0, i)),
        out_shape=jax.ShapeDtypeStruct((_OUT, _B), jnp.float32),
    )(w, pooled1, bias)
    return pl.pallas_call(
        _mm_body2,
        grid=(_NBLK,),
        in_specs=[
            pl.BlockSpec((_OUT, _D), lambda i: (0, 0)),
            pl.BlockSpec((_BM, _D), lambda i: (i, 0)),
            pl.BlockSpec((_OUT, 1), lambda i: (0, 0)),
            pl.BlockSpec(memory_space=pl.ANY),
        ],
        out_specs=pl.BlockSpec((_OUT, _BM), lambda i: (0, i + _NBLK)),
        out_shape=jax.ShapeDtypeStruct((_OUT, _B), jnp.float32),
        input_output_aliases={3: 0},
    )(w, pooled2, bias, part)


def kernel(x, emb_table, fc_w, fc_b):
    tab = _transpose_table(emb_table.T).reshape(_VOCAB, _D)
    pooled1 = _pool(x, tab, 0)
    pooled2 = _pool(x, tab, 1)
    out_t = _matmul_t(pooled1, pooled2, fc_w, fc_b.reshape(_OUT, 1))
    return out_t.T


# speedup vs baseline: 1.0398x; 1.0398x over previous
import functools

import jax
import jax.numpy as jnp
from jax import lax
from jax.experimental import pallas as pl
from jax.experimental.pallas import tpu as pltpu
from jax.experimental.pallas import tpu_sc as plsc

_VOCAB = 1000000
_D = 32
_OUT = 1000
_B = 16384
_H = 200

_NC = 2
_NS = 16
_NW = _NC * _NS

_TCOLS = _VOCAB // 128
_TAIL = _VOCAB - _TCOLS * 128


_NDB = _TCOLS // 2
_BW = 256


def _transpose_body(tabt_hbm, out_hbm, inb, outc, isem, osem):
    wid = lax.axis_index("s") * _NC + lax.axis_index("c")
    n_t = jnp.where(wid < _NDB - 32 * (_NDB // 32), _NDB // 32 + 1,
                    _NDB // 32)
    iota16 = lax.iota(jnp.int32, 16)
    cvecs = [iota16 + 16 * k for k in range(8)]
    svecs = [iota16 * 32 + 512 * k for k in range(8)]

    def in_copy(t, p):
        c = wid + 32 * jnp.minimum(t, n_t - 1)
        c0 = pl.multiple_of(c * _BW, 128)
        for r in range(4):
            pltpu.async_copy(
                tabt_hbm.at[pl.ds(8 * r, 8), pl.ds(c0, _BW)],
                inb.at[p, r], isem)

    def wait_in(p):
        for r in range(4):
            pltpu.make_async_copy(
                tabt_hbm.at[pl.ds(0, 8), pl.ds(0, _BW)],
                inb.at[p, r], isem).wait()

    def out_copy(t, p):
        c = wid + 32 * jnp.minimum(t, n_t - 1)
        pltpu.async_copy(outc.at[p],
                         out_hbm.at[pl.ds(c * (_BW * _D), _BW * _D)], osem)

    def wait_out(p):
        pltpu.make_async_copy(outc.at[0],
                              out_hbm.at[pl.ds(0, _BW * _D)], osem).wait()

    def transpose_block(p, nks):
        idx_p = jnp.full((16,), p, dtype=jnp.int32)
        for s, nk in enumerate(nks):

            def diag_body(d0, carry, s=s, nk=nk):
                dvec = (d0 + iota16) & 31
                d8 = dvec >> 3
                d7 = dvec & 7
                vs = [plsc.load_gather(
                    inb, [idx_p, d8, d7, cvecs[k] + 128 * s])
                    for k in range(nk)]
                for k in range(nk):
                    plsc.store_scatter(
                        outc, [idx_p, svecs[k] + dvec + 4096 * s], vs[k])
                return carry

            lax.fori_loop(0, 32, diag_body, 0)

    for t0 in range(4):
        in_copy(t0, t0)
    for t0 in range(4):
        wait_in(t0)
        transpose_block(t0, (8, 8))
        in_copy(t0 + 4, t0)
        out_copy(t0, t0)

    n_t4 = ((n_t + 3) // 4) * 4

    def steady(j, carry):
        for u in range(4):
            t = 4 * j + u
            wait_in(u)
            wait_out(u)
            transpose_block(u, (8, 8))
            in_copy(t + 4, u)
            out_copy(t, u)
        return carry

    lax.fori_loop(1, n_t4 // 4, steady, 0)
    for t0 in range(4):
        wait_out(t0)
        wait_in(t0)

    @pl.when(wid == _NW - 1)
    def _tail():
        tail_c0 = pl.multiple_of(
            lax.max(wid * 128, jnp.int32((_TCOLS - 1) * 128)), 128)
        for r in range(4):
            pltpu.sync_copy(
                tabt_hbm.at[pl.ds(8 * r, 8), pl.ds(tail_c0, _BW)],
                inb.at[0, r])
        transpose_block(0, (8, _TAIL // 16))
        n_tail = (128 + _TAIL) * _D
        pltpu.sync_copy(outc.at[0, pl.ds(0, n_tail)],
                        out_hbm.at[pl.ds((_TCOLS - 1) * 128 * _D, n_tail)])


def _transpose_table(tabt):
    mesh = plsc.VectorSubcoreMesh(core_axis_name="c", subcore_axis_name="s")
    return pl.kernel(
        _transpose_body,
        out_type=jax.ShapeDtypeStruct((_VOCAB * _D,), jnp.float32),
        mesh=mesh,
        scratch_types=[
            pltpu.VMEM((4, 4, 8, _BW), jnp.float32),
            pltpu.VMEM((4, _BW * _D), jnp.float32),
            pltpu.SemaphoreType.DMA,
            pltpu.SemaphoreType.DMA,
        ],
        compiler_params=pltpu.CompilerParams(use_tc_tiling_on_sc=True,
                                             needs_layout_passes=False),
    )(tabt)


_RPW = _B // _NW
_C = 8
_G1 = 104
_G2 = _H - _G1
_NCHUNK = _RPW // _C
_NPAIR = _NCHUNK // 2


def _fire_gathers(tab, idx_b, rows_b, sem):
    for c in range(_C):
        pltpu.async_copy(
            tab.at[idx_b.at[c, pl.ds(0, _G1)]],
            rows_b.at[pl.ds(c * _H, _G1)], sem)
        pltpu.async_copy(
            tab.at[idx_b.at[c, pl.ds(_G1, _G2)]],
            rows_b.at[pl.ds(c * _H + _G1, _G2)], sem)


def _wait_gathers(tab, idx_b, rows_b, sem):
    for c in range(_C):
        pltpu.make_async_copy(
            tab.at[idx_b.at[c, pl.ds(0, _G1)]],
            rows_b.at[pl.ds(c * _H, _G1)], sem).wait()
        pltpu.make_async_copy(
            tab.at[idx_b.at[c, pl.ds(_G1, _G2)]],
            rows_b.at[pl.ds(c * _H + _G1, _G2)], sem).wait()


def _wait_idx(x_hbm, idx_b, sem):
    pltpu.make_async_copy(x_hbm.at[pl.ds(0, _C)], idx_b, sem).wait()


def _reduce_chunk(rows_b, stage_v, chunk):
    inv = jnp.float32(1.0 / _H)
    for c in range(_C):
        s0 = c * _H

        def body(j, accs, s0=s0):
            a0, a1, b0, b1 = accs
            r = s0 + 4 * j
            a0 = a0 + rows_b[r, pl.ds(0, 16)]
            a1 = a1 + rows_b[r, pl.ds(16, 16)]
            b0 = b0 + rows_b[r + 1, pl.ds(0, 16)]
            b1 = b1 + rows_b[r + 1, pl.ds(16, 16)]
            a0 = a0 + rows_b[r + 2, pl.ds(0, 16)]
            a1 = a1 + rows_b[r + 2, pl.ds(16, 16)]
            b0 = b0 + rows_b[r + 3, pl.ds(0, 16)]
            b1 = b1 + rows_b[r + 3, pl.ds(16, 16)]
            return (a0, a1, b0, b1)

        z = jnp.zeros((16,), jnp.float32)
        a0, a1, b0, b1 = lax.fori_loop(0, _H // 4, body, (z, z, z, z))
        slot = chunk * _C + c
        stage_v[slot, pl.ds(0, 16)] = (a0 + b0) * inv
        stage_v[slot, pl.ds(16, 16)] = (a1 + b1) * inv


def _pool_body(x_hbm, tab_hbm, out_hbm,
               idx0, idx1, rows0, rows1, stage_v, isem, gsem0, gsem1):
    wid = lax.axis_index("s") * _NC + lax.axis_index("c")
    row0 = wid * _RPW

    def idx_copy(t, dst):
        base = row0 + jnp.minimum(t, _NCHUNK - 1) * _C
        pltpu.async_copy(x_hbm.at[pl.ds(base, _C)], dst, isem)

    idx_copy(0, idx0)
    _wait_idx(x_hbm, idx0, isem)
    _fire_gathers(tab_hbm, idx0, rows0, gsem0)
    idx_copy(1, idx1)

    def pair_body(p, carry):
        a = 2 * p
        _wait_idx(x_hbm, idx1, isem)
        _fire_gathers(tab_hbm, idx1, rows1, gsem1)
        _wait_gathers(tab_hbm, idx0, rows0, gsem0)
        idx_copy(a + 2, idx0)
        _reduce_chunk(rows0, stage_v, a)
        _wait_idx(x_hbm, idx0, isem)
        _fire_gathers(tab_hbm, idx0, rows0, gsem0)
        _wait_gathers(tab_hbm, idx1, rows1, gsem1)
        idx_copy(a + 3, idx1)
        _reduce_chunk(rows1, stage_v, a + 1)
        return carry

    lax.fori_loop(0, _NPAIR, pair_body, 0)

    _wait_idx(x_hbm, idx1, isem)
    _wait_gathers(tab_hbm, idx0, rows0, gsem0)
    pltpu.sync_copy(stage_v, out_hbm.at[pl.ds(row0, _RPW)])


def _pool(x, tab):
    mesh = plsc.VectorSubcoreMesh(core_axis_name="c", subcore_axis_name="s")
    return pl.kernel(
        _pool_body,
        out_type=jax.ShapeDtypeStruct((_B, _D), jnp.float32),
        mesh=mesh,
        scratch_types=[
            pltpu.VMEM((_C, _H), jnp.int32),
            pltpu.VMEM((_C, _H), jnp.int32),
            pltpu.VMEM((_C * _H, _D), jnp.float32),
            pltpu.VMEM((_C * _H, _D), jnp.float32),
            pltpu.VMEM((_RPW, _D), jnp.float32),
            pltpu.SemaphoreType.DMA,
            pltpu.SemaphoreType.DMA,
            pltpu.SemaphoreType.DMA,
        ],
        compiler_params=pltpu.CompilerParams(use_tc_tiling_on_sc=False),
    )(x, tab)


_BM = 2048


def _mm_body(w_ref, p_ref, b_ref, o_ref):
    o_ref[...] = (
        lax.dot_general(
            w_ref[...], p_ref[...],
            dimension_numbers=(((1,), (1,)), ((), ())),
            preferred_element_type=jnp.float32,
        )
        + b_ref[...]
    )


def _matmul_t(pooled, w, bias):
    return pl.pallas_call(
        _mm_body,
        grid=(_B // _BM,),
        in_specs=[
            pl.BlockSpec((_OUT, _D), lambda i: (0, 0)),
            pl.BlockSpec((_BM, _D), lambda i: (i, 0)),
            pl.BlockSpec((_OUT, 1), lambda i: (0, 0)),
        ],
        out_specs=pl.BlockSpec((_OUT, _BM), lambda i: (0, i)),
        out_shape=jax.ShapeDtypeStruct((_OUT, _B), jnp.float32),
    )(w, pooled, bias)


def kernel(x, emb_table, fc_w, fc_b):
    tab = _transpose_table(emb_table.T).reshape(_VOCAB, _D)
    pooled = _pool(x, tab)
    out_t = _matmul_t(pooled, fc_w, fc_b.reshape(_OUT, 1))
    return out_t.T
